# baseline (speedup 1.0000x reference)
import jax
import jax.numpy as jnp
from jax.experimental import pallas as pl
from jax.experimental.pallas import tpu as pltpu

DIM = 2048
IMG = 784
LEVELS = 256
NUM_CLASSES = 10
BATCH = 128


def _hdc_body(x_ref, pos_ref, vt_ref, am_ref, out_ref):
    xrow = x_ref[0]
    idx = jnp.clip(jnp.round(xrow * (LEVELS - 1)), 0.0, LEVELS - 1.0).astype(jnp.int32)
    lvl = jax.lax.broadcasted_iota(jnp.int32, (LEVELS, IMG), 0)
    onehot = (lvl == idx).astype(jnp.int8)
    pos = pos_ref[...]
    h = jax.lax.dot_general(
        onehot, pos, (((1,), (0,)), ((), ())),
        preferred_element_type=jnp.int32)
    enc = jnp.sum(h.astype(jnp.float32) * vt_ref[...], axis=0, keepdims=True)
    am = am_ref[...]
    dots = jax.lax.dot_general(
        enc, am, (((1,), (1,)), ((), ())),
        preferred_element_type=jnp.float32)
    ne = jnp.sqrt(jnp.sum(enc * enc)) + 1e-12
    na = jnp.sqrt(jnp.sum(am * am, axis=1)).reshape(1, NUM_CLASSES) + 1e-12
    out_ref[0] = dots / ne / na


def kernel(x, position, value_table, am):
    flat = x.reshape(BATCH, 1, IMG)
    pos_bf = position.astype(jnp.int8)
    out = pl.pallas_call(
        _hdc_body,
        grid=(BATCH,),
        in_specs=[
            pl.BlockSpec((1, 1, IMG), lambda i: (i, 0, 0)),
            pl.BlockSpec((IMG, DIM), lambda i: (0, 0)),
            pl.BlockSpec((LEVELS, DIM), lambda i: (0, 0)),
            pl.BlockSpec((NUM_CLASSES, DIM), lambda i: (0, 0)),
        ],
        out_specs=pl.BlockSpec((1, 1, NUM_CLASSES), lambda i: (i, 0, 0)),
        out_shape=jax.ShapeDtypeStruct((BATCH, 1, NUM_CLASSES), jnp.float32),
    )(flat, pos_bf, value_table, am)
    return out.reshape(BATCH, NUM_CLASSES)


# speedup vs baseline: 1.0609x; 1.0609x over previous
import jax
import jax.numpy as jnp
from jax.experimental import pallas as pl
from jax.experimental.pallas import tpu as pltpu

DIM = 2048
IMG = 784
LEVELS = 256
NUM_CLASSES = 10
BATCH = 128
ITEMS = 2


def _hdc_body(x_ref, pos_ref, vt_ref, am_ref, out_ref):
    xb = x_ref[0]
    idx = jnp.clip(jnp.round(xb * (LEVELS - 1)), 0.0, LEVELS - 1.0).astype(jnp.int32)
    lvl = jax.lax.broadcasted_iota(jnp.int32, (LEVELS, IMG), 0)
    pos = pos_ref[...]
    vt = vt_ref[...]
    am = am_ref[...]
    na = jnp.sqrt(jnp.sum(am * am, axis=1)).reshape(1, NUM_CLASSES) + 1e-12
    for j in range(ITEMS):
        onehot = (lvl == idx[j:j + 1, :]).astype(jnp.bfloat16)
        h = jax.lax.dot_general(
            onehot, pos, (((1,), (0,)), ((), ())),
            preferred_element_type=jnp.float32)
        enc = jnp.sum(h * vt, axis=0, keepdims=True)
        dots = jax.lax.dot_general(
            enc, am, (((1,), (1,)), ((), ())),
            preferred_element_type=jnp.float32)
        ne = jnp.sqrt(jnp.sum(enc * enc)) + 1e-12
        out_ref[0, j:j + 1, :] = dots / ne / na


def kernel(x, position, value_table, am):
    flat = x.reshape(BATCH // ITEMS, ITEMS, IMG)
    pos_bf = position.astype(jnp.bfloat16)
    out = pl.pallas_call(
        _hdc_body,
        grid=(BATCH // ITEMS,),
        in_specs=[
            pl.BlockSpec((1, ITEMS, IMG), lambda i: (i, 0, 0)),
            pl.BlockSpec((IMG, DIM), lambda i: (0, 0)),
            pl.BlockSpec((LEVELS, DIM), lambda i: (0, 0)),
            pl.BlockSpec((NUM_CLASSES, DIM), lambda i: (0, 0)),
        ],
        out_specs=pl.BlockSpec((1, ITEMS, NUM_CLASSES), lambda i: (i, 0, 0)),
        out_shape=jax.ShapeDtypeStruct((BATCH // ITEMS, ITEMS, NUM_CLASSES), jnp.float32),
    )(flat, pos_bf, value_table, am)
    return out.reshape(BATCH, NUM_CLASSES)


# speedup vs baseline: 1.2626x; 1.1901x over previous
import jax
import jax.numpy as jnp
from jax.experimental import pallas as pl
from jax.experimental.pallas import tpu as pltpu

DIM = 2048
IMG = 784
LEVELS = 256
NUM_CLASSES = 10
BATCH = 128
ITEMS = 8


def _hdc_body(x_ref, pos_ref, vt_ref, am_ref, out_ref, enc_ref):
    xb = x_ref[0]
    idx = jnp.clip(jnp.round(xb * (LEVELS - 1)), 0.0, LEVELS - 1.0).astype(jnp.int32)
    lvl = jax.lax.broadcasted_iota(jnp.int32, (LEVELS, IMG), 0)
    pos = pos_ref[...]
    vt = vt_ref[...]
    for j in range(ITEMS):
        onehot = (lvl == idx[j:j + 1, :]).astype(jnp.bfloat16)
        h = jax.lax.dot_general(
            onehot, pos, (((1,), (0,)), ((), ())),
            preferred_element_type=jnp.float32)
        enc_ref[j:j + 1, :] = jnp.sum(h * vt, axis=0, keepdims=True)
    enc = enc_ref[...]
    am = am_ref[...]
    dots = jax.lax.dot_general(
        enc, am, (((1,), (1,)), ((), ())),
        preferred_element_type=jnp.float32)
    ne = jnp.sqrt(jnp.sum(enc * enc, axis=1, keepdims=True)) + 1e-12
    na = jnp.sqrt(jnp.sum(am * am, axis=1)).reshape(1, NUM_CLASSES) + 1e-12
    out_ref[0] = dots / ne / na


def kernel(x, position, value_table, am):
    flat = x.reshape(BATCH // ITEMS, ITEMS, IMG)
    pos_bf = position.astype(jnp.bfloat16)
    out = pl.pallas_call(
        _hdc_body,
        grid=(BATCH // ITEMS,),
        in_specs=[
            pl.BlockSpec((1, ITEMS, IMG), lambda i: (i, 0, 0)),
            pl.BlockSpec((IMG, DIM), lambda i: (0, 0)),
            pl.BlockSpec((LEVELS, DIM), lambda i: (0, 0)),
            pl.BlockSpec((NUM_CLASSES, DIM), lambda i: (0, 0)),
        ],
        out_specs=pl.BlockSpec((1, ITEMS, NUM_CLASSES), lambda i: (i, 0, 0)),
        out_shape=jax.ShapeDtypeStruct((BATCH // ITEMS, ITEMS, NUM_CLASSES), jnp.float32),
        scratch_shapes=[pltpu.VMEM((ITEMS, DIM), jnp.float32)],
    )(flat, pos_bf, value_table, am)
    return out.reshape(BATCH, NUM_CLASSES)
